# baseline (speedup 1.0000x reference)
import jax
import jax.numpy as jnp
from jax.experimental import pallas as pl
from jax.experimental.pallas import tpu as pltpu

_B = 4096
_D = 256
_BB = 64
_EPS = 1e-5


def _ln(x, g, b):
    ones = jnp.ones((x.shape[1], 1), jnp.float32)
    s1 = jnp.dot(x, ones, preferred_element_type=jnp.float32)
    s2 = jnp.dot(x * x, ones, preferred_element_type=jnp.float32)
    m = s1 * (1.0 / _D)
    var = s2 * (1.0 / _D) - m * m
    k = jax.lax.rsqrt(var + _EPS)
    return (x * k - m * k) * g + b


def _body(cid_h, enh_h, ed_h, seal_h, f0_h, f1_h, colsp_c, vals_c, ids_m,
          feats, Th, Tc, Tm, run_W, vecs,
          hand_out, run_out, ctx_out, mod_out):
    v = vecs[...]
    run_b, run_g, run_be = v[0:1], v[1:2], v[2:3]
    hl_g, hl_be = v[3:4], v[4:5]
    mod_g, mod_be = v[5:6], v[6:7]
    hand_g, hand_be = v[7:8], v[8:9]
    deck_g, deck_be = v[9:10], v[10:11]

    cid = cid_h[...]
    rows = cid.shape[0]
    mf = (cid >= 0).astype(jnp.float32)
    c0 = jnp.maximum(cid, 0)
    c1 = 52 + 4 * enh_h[...] + ed_h[...]
    c2 = 88 + seal_h[...]
    iota = jax.lax.broadcasted_iota(jnp.int32, (rows, 95), 1)
    eq = (iota == c0) | (iota == c1) | (iota == c2)
    acc = jnp.where(eq, mf, 0.0)
    acc = acc + jnp.where(iota == 93, f0_h[...] * mf, 0.0)
    acc = acc + jnp.where(iota == 94, f1_h[...] * mf, 0.0)
    y = jnp.dot(acc, Th[...], preferred_element_type=jnp.float32)
    hand_out[...] = _ln(y, hand_g, hand_be)

    p = colsp_c[...]
    rows = p.shape[0]
    c0 = p & 127
    c1 = (p >> 7) & 127
    c2 = (p >> 14) & 127
    row = jax.lax.broadcasted_iota(jnp.int32, (rows, 1), 0)
    sel = (row % 64) < 12
    tf = sel.astype(jnp.float32)
    va = vals_c[...]
    a, b2 = va[:, 0:1], va[:, 1:2]
    v0 = jnp.where(sel, 1.0, a)
    iota = jax.lax.broadcasted_iota(jnp.int32, (rows, 108), 1)
    acc = jnp.where(iota == c0, v0, 0.0)
    acc = acc + jnp.where(iota == c1, a, 0.0)
    acc = acc + jnp.where(iota == c2, b2, 0.0)
    acc = acc + jnp.where(iota == 14, tf, 0.0)
    y = jnp.dot(acc, Tc[...], preferred_element_type=jnp.float32)
    g = jnp.where(sel, hl_g, deck_g)
    b = jnp.where(sel, hl_be, deck_be)
    ctx_out[...] = _ln(y, g, b)

    ids = ids_m[...]
    rows = ids.shape[0]
    vm = (ids != 0).astype(jnp.float32)
    row = jax.lax.broadcasted_iota(jnp.int32, (rows, 1), 0)
    pos = 179 + row % 11
    iota = jax.lax.broadcasted_iota(jnp.int32, (rows, 190), 1)
    acc = jnp.where(iota == ids, vm, 0.0)
    acc = acc + jnp.where(iota == pos, 1.0, 0.0)
    y = jnp.dot(acc, Tm[...], preferred_element_type=jnp.float32)
    mod_out[...] = _ln(y, mod_g, mod_be)

    y = jnp.dot(feats[...], run_W[...],
                preferred_element_type=jnp.float32) + run_b
    run_out[...] = _ln(y, run_g, run_be)


def kernel(hand_card_ids, hand_card_enhancements, hand_card_editions,
           hand_card_seals, hand_is_face_down, hand_is_debuffed,
           deck_card_ids, deck_card_enhancements, deck_card_editions,
           deck_card_seals, hands_remaining, discards_remaining, money,
           current_score, target_score, hand_levels, boss_id, boss_is_active,
           joker_ids, joker_is_empty, h_rank, h_suit, h_enh, h_ed, h_seal,
           d_rank, d_suit, d_enh, d_ed, d_seal, Wf, run_W, run_b, run_g,
           run_be, hl_type, hl_W, hl_b, hl_g, hl_be, mod_emb_t, mod_pos,
           mod_g, mod_be, hand_g, hand_be, deck_g, deck_be):
    i32 = lambda x: x.astype(jnp.int32)
    f32 = lambda x: x.astype(jnp.float32)
    Bn = hand_card_ids.shape[0]

    cid52 = jnp.arange(52)
    Thcard = h_rank[cid52 // 4] + h_suit[cid52 % 4]
    Tdcard = d_rank[cid52 // 4] + d_suit[cid52 % 4]
    Thee = (h_enh[:, None, :] + h_ed[None, :, :]).reshape(36, _D)
    Tdee = (d_enh[:, None, :] + d_ed[None, :, :]).reshape(36, _D)
    Th = jnp.concatenate([Thcard, Thee, h_seal, Wf], axis=0)
    Tc = jnp.concatenate([hl_type, hl_W, hl_b[None, :], Tdcard, Tdee,
                          d_seal], axis=0)
    Tm = jnp.concatenate([mod_emb_t, mod_pos], axis=0)

    hand_cid = i32(hand_card_ids)
    hmask = hand_cid >= 0
    cid_h = hand_cid.reshape(Bn * 16, 1)
    enh_h = i32(hand_card_enhancements).reshape(Bn * 16, 1)
    ed_h = i32(hand_card_editions).reshape(Bn * 16, 1)
    seal_h = i32(hand_card_seals).reshape(Bn * 16, 1)
    f0_h = f32(hand_is_face_down).reshape(Bn * 16, 1)
    f1_h = f32(hand_is_debuffed).reshape(Bn * 16, 1)

    hl_ids = i32(hand_levels[:, :, 0])
    hlf0 = f32(hand_levels[:, :, 2])
    hlf1 = f32(hand_levels[:, :, 3])
    packed_hl = hl_ids | (12 << 7) | (13 << 14)
    deck_cid = i32(deck_card_ids)
    dmask = deck_cid >= 0
    dmf = f32(dmask)
    dsafe = jnp.maximum(deck_cid, 0)
    packed_dk = ((15 + dsafe)
                 | ((67 + 4 * i32(deck_card_enhancements)
                     + i32(deck_card_editions)) << 7)
                 | ((103 + i32(deck_card_seals)) << 14))
    colsp_c = jnp.concatenate([packed_hl, packed_dk],
                              axis=1).reshape(Bn * 64, 1)
    vals_c = jnp.concatenate(
        [jnp.stack([hlf0, hlf1], axis=-1),
         jnp.stack([dmf, dmf], axis=-1)], axis=1).reshape(Bn * 64, 2)

    has_boss = boss_is_active.astype(bool)
    jok = i32(joker_ids)
    mod_ids = jnp.where(has_boss[:, None],
                        jnp.concatenate([(i32(boss_id) + 150)[:, None], jok],
                                        axis=1),
                        jnp.concatenate([jok, jnp.zeros((Bn, 1), jnp.int32)],
                                        axis=1))
    ids_m = mod_ids.reshape(Bn * 11, 1)

    mf = f32(money)
    feats = jnp.stack([f32(hands_remaining), f32(discards_remaining),
                       jnp.sign(mf) * jnp.log1p(jnp.abs(mf)),
                       jnp.log1p(f32(current_score)),
                       jnp.log1p(f32(target_score))], axis=-1)

    vecs = jnp.stack([run_b, run_g, run_be, hl_g, hl_be, mod_g, mod_be,
                      hand_g, hand_be, deck_g, deck_be], axis=0)

    grid = (Bn // _BB,)
    rspec = lambda r, n: pl.BlockSpec((r * _BB, n), lambda i: (i, 0))
    tspec = lambda r: pl.BlockSpec((r, _D), lambda i: (0, 0))

    hand2, run2, ctx2, mod2 = pl.pallas_call(
        _body,
        grid=grid,
        in_specs=[
            rspec(16, 1), rspec(16, 1), rspec(16, 1), rspec(16, 1),
            rspec(16, 1), rspec(16, 1),
            rspec(64, 1), rspec(64, 2),
            rspec(11, 1),
            rspec(1, 5),
            tspec(95), tspec(108), tspec(190), tspec(5), tspec(11),
        ],
        out_specs=[rspec(16, _D), rspec(1, _D), rspec(64, _D),
                   rspec(11, _D)],
        out_shape=[
            jax.ShapeDtypeStruct((Bn * 16, _D), jnp.float32),
            jax.ShapeDtypeStruct((Bn, _D), jnp.float32),
            jax.ShapeDtypeStruct((Bn * 64, _D), jnp.float32),
            jax.ShapeDtypeStruct((Bn * 11, _D), jnp.float32),
        ],
        compiler_params=pltpu.CompilerParams(
            dimension_semantics=("arbitrary",)),
    )(cid_h, enh_h, ed_h, seal_h, f0_h, f1_h, colsp_c, vals_c, ids_m,
      feats, Th, Tc, Tm, run_W, vecs)

    hand_toks = hand2.reshape(Bn, 16, _D)
    run_tok = run2.reshape(Bn, 1, _D)
    ctx_seq = ctx2.reshape(Bn, 64, _D)
    mod_seq = mod2.reshape(Bn, 11, _D)

    ctx_mask = jnp.concatenate([jnp.ones((Bn, 12), dtype=bool), dmask],
                               axis=1)
    joker_real = joker_is_empty == 0
    mod_mask = jnp.where(has_boss[:, None],
                         jnp.concatenate(
                             [jnp.ones((Bn, 1), dtype=bool), joker_real],
                             axis=1),
                         jnp.concatenate(
                             [joker_real, jnp.zeros((Bn, 1), dtype=bool)],
                             axis=1))
    no_mod = ~jnp.any(mod_mask, axis=1)
    mod_mask = mod_mask.at[:, 0].set(mod_mask[:, 0] | no_mod)

    return (hand_toks, hmask, run_tok, ctx_seq, ctx_mask, mod_seq, mod_mask)


# speedup vs baseline: 1.0017x; 1.0017x over previous
import jax
import jax.numpy as jnp
from jax.experimental import pallas as pl
from jax.experimental.pallas import tpu as pltpu

_B = 4096
_D = 256
_BB = 64
_EPS = 1e-5


def _ln(x, g, b):
    ones = jnp.ones((x.shape[1], 1), jnp.float32)
    s1 = jnp.dot(x, ones, preferred_element_type=jnp.float32)
    s2 = jnp.dot(x * x, ones, preferred_element_type=jnp.float32)
    m = s1 * (1.0 / _D)
    var = s2 * (1.0 / _D) - m * m
    k = jax.lax.rsqrt(var + _EPS)
    return (x * k - m * k) * g + b


def _body(cid_h, enh_h, ed_h, seal_h, f0_h, f1_h, colsp_c, vals_c, ids_m,
          feats, Th, Tc, Tm, run_W, vecs,
          hand_out, run_out, ctx_out, mod_out):
    v = vecs[...]
    run_b, run_g, run_be = v[0:1], v[1:2], v[2:3]
    hl_g, hl_be = v[3:4], v[4:5]
    mod_g, mod_be = v[5:6], v[6:7]
    hand_g, hand_be = v[7:8], v[8:9]
    deck_g, deck_be = v[9:10], v[10:11]

    cid = cid_h[...]
    rows = cid.shape[0]
    mf = (cid >= 0).astype(jnp.float32)
    c0 = jnp.maximum(cid, 0)
    c1 = 52 + 4 * enh_h[...] + ed_h[...]
    c2 = 88 + seal_h[...]
    iota = jax.lax.broadcasted_iota(jnp.int32, (rows, 95), 1)
    eq = (iota == c0) | (iota == c1) | (iota == c2)
    acc = jnp.where(eq, mf, 0.0)
    acc = acc + jnp.where(iota == 93, f0_h[...] * mf, 0.0)
    acc = acc + jnp.where(iota == 94, f1_h[...] * mf, 0.0)
    y = jnp.dot(acc, Th[...], preferred_element_type=jnp.float32)
    hand_out[...] = _ln(y, hand_g, hand_be)

    p = colsp_c[...]
    rows = p.shape[0]
    c0 = p & 127
    c1 = (p >> 7) & 127
    c2 = (p >> 14) & 127
    row = jax.lax.broadcasted_iota(jnp.int32, (rows, 1), 0)
    sel = (row % 64) < 12
    tf = sel.astype(jnp.float32)
    va = vals_c[...]
    a, b2 = va[:, 0:1], va[:, 1:2]
    v0 = jnp.where(sel, 1.0, a)
    iota = jax.lax.broadcasted_iota(jnp.int32, (rows, 108), 1)
    acc = jnp.where(iota == c0, v0, 0.0)
    acc = acc + jnp.where(iota == c1, a, 0.0)
    acc = acc + jnp.where(iota == c2, b2, 0.0)
    acc = acc + jnp.where(iota == 14, tf, 0.0)
    y = jnp.dot(acc, Tc[...], preferred_element_type=jnp.float32)
    g = jnp.where(sel, hl_g, deck_g)
    b = jnp.where(sel, hl_be, deck_be)
    ctx_out[...] = _ln(y, g, b)

    ids = ids_m[...]
    rows = ids.shape[0]
    vm = (ids != 0).astype(jnp.float32)
    row = jax.lax.broadcasted_iota(jnp.int32, (rows, 1), 0)
    pos = 179 + row % 11
    iota = jax.lax.broadcasted_iota(jnp.int32, (rows, 190), 1)
    acc = jnp.where(iota == ids, vm, 0.0)
    acc = acc + jnp.where(iota == pos, 1.0, 0.0)
    y = jnp.dot(acc, Tm[...], preferred_element_type=jnp.float32)
    mod_out[...] = _ln(y, mod_g, mod_be)

    y = jnp.dot(feats[...], run_W[...],
                preferred_element_type=jnp.float32) + run_b
    run_out[...] = _ln(y, run_g, run_be)


def kernel(hand_card_ids, hand_card_enhancements, hand_card_editions,
           hand_card_seals, hand_is_face_down, hand_is_debuffed,
           deck_card_ids, deck_card_enhancements, deck_card_editions,
           deck_card_seals, hands_remaining, discards_remaining, money,
           current_score, target_score, hand_levels, boss_id, boss_is_active,
           joker_ids, joker_is_empty, h_rank, h_suit, h_enh, h_ed, h_seal,
           d_rank, d_suit, d_enh, d_ed, d_seal, Wf, run_W, run_b, run_g,
           run_be, hl_type, hl_W, hl_b, hl_g, hl_be, mod_emb_t, mod_pos,
           mod_g, mod_be, hand_g, hand_be, deck_g, deck_be):
    i32 = lambda x: x.astype(jnp.int32)
    f32 = lambda x: x.astype(jnp.float32)
    Bn = hand_card_ids.shape[0]

    cid52 = jnp.arange(52)
    Thcard = h_rank[cid52 // 4] + h_suit[cid52 % 4]
    Tdcard = d_rank[cid52 // 4] + d_suit[cid52 % 4]
    Thee = (h_enh[:, None, :] + h_ed[None, :, :]).reshape(36, _D)
    Tdee = (d_enh[:, None, :] + d_ed[None, :, :]).reshape(36, _D)
    Th = jnp.concatenate([Thcard, Thee, h_seal, Wf], axis=0)
    Tc = jnp.concatenate([hl_type, hl_W, hl_b[None, :], Tdcard, Tdee,
                          d_seal], axis=0)
    Tm = jnp.concatenate([mod_emb_t, mod_pos], axis=0)

    hand_cid = i32(hand_card_ids)
    hmask = hand_cid >= 0
    cid_h = hand_cid.reshape(Bn * 16, 1)
    enh_h = i32(hand_card_enhancements).reshape(Bn * 16, 1)
    ed_h = i32(hand_card_editions).reshape(Bn * 16, 1)
    seal_h = i32(hand_card_seals).reshape(Bn * 16, 1)
    f0_h = f32(hand_is_face_down).reshape(Bn * 16, 1)
    f1_h = f32(hand_is_debuffed).reshape(Bn * 16, 1)

    hl_ids = i32(hand_levels[:, :, 0])
    hlf0 = f32(hand_levels[:, :, 2])
    hlf1 = f32(hand_levels[:, :, 3])
    packed_hl = hl_ids | (12 << 7) | (13 << 14)
    deck_cid = i32(deck_card_ids)
    dmask = deck_cid >= 0
    dmf = f32(dmask)
    dsafe = jnp.maximum(deck_cid, 0)
    packed_dk = ((15 + dsafe)
                 | ((67 + 4 * i32(deck_card_enhancements)
                     + i32(deck_card_editions)) << 7)
                 | ((103 + i32(deck_card_seals)) << 14))
    colsp_c = jnp.concatenate([packed_hl, packed_dk],
                              axis=1).reshape(Bn * 64, 1)
    vals_c = jnp.concatenate(
        [jnp.stack([hlf0, hlf1], axis=-1),
         jnp.stack([dmf, dmf], axis=-1)], axis=1).reshape(Bn * 64, 2)

    has_boss = boss_is_active.astype(bool)
    jok = i32(joker_ids)
    mod_ids = jnp.where(has_boss[:, None],
                        jnp.concatenate([(i32(boss_id) + 150)[:, None], jok],
                                        axis=1),
                        jnp.concatenate([jok, jnp.zeros((Bn, 1), jnp.int32)],
                                        axis=1))
    ids_m = mod_ids.reshape(Bn * 11, 1)

    mf = f32(money)
    feats = jnp.stack([f32(hands_remaining), f32(discards_remaining),
                       jnp.sign(mf) * jnp.log1p(jnp.abs(mf)),
                       jnp.log1p(f32(current_score)),
                       jnp.log1p(f32(target_score))], axis=-1)

    vecs = jnp.stack([run_b, run_g, run_be, hl_g, hl_be, mod_g, mod_be,
                      hand_g, hand_be, deck_g, deck_be], axis=0)

    grid = (Bn // _BB,)
    rspec = lambda r, n: pl.BlockSpec((r * _BB, n), lambda i: (i, 0))
    tspec = lambda r: pl.BlockSpec((r, _D), lambda i: (0, 0))

    hand2, run2, ctx2, mod2 = pl.pallas_call(
        _body,
        grid=grid,
        in_specs=[
            rspec(16, 1), rspec(16, 1), rspec(16, 1), rspec(16, 1),
            rspec(16, 1), rspec(16, 1),
            rspec(64, 1), rspec(64, 2),
            rspec(11, 1),
            rspec(1, 5),
            tspec(95), tspec(108), tspec(190), tspec(5), tspec(11),
        ],
        out_specs=[rspec(16, _D), rspec(1, _D), rspec(64, _D),
                   rspec(11, _D)],
        out_shape=[
            jax.ShapeDtypeStruct((Bn * 16, _D), jnp.float32),
            jax.ShapeDtypeStruct((Bn, _D), jnp.float32),
            jax.ShapeDtypeStruct((Bn * 64, _D), jnp.float32),
            jax.ShapeDtypeStruct((Bn * 11, _D), jnp.float32),
        ],
        compiler_params=pltpu.CompilerParams(
            dimension_semantics=("parallel",)),
    )(cid_h, enh_h, ed_h, seal_h, f0_h, f1_h, colsp_c, vals_c, ids_m,
      feats, Th, Tc, Tm, run_W, vecs)

    hand_toks = hand2.reshape(Bn, 16, _D)
    run_tok = run2.reshape(Bn, 1, _D)
    ctx_seq = ctx2.reshape(Bn, 64, _D)
    mod_seq = mod2.reshape(Bn, 11, _D)

    ctx_mask = jnp.concatenate([jnp.ones((Bn, 12), dtype=bool), dmask],
                               axis=1)
    joker_real = joker_is_empty == 0
    mod_mask = jnp.where(has_boss[:, None],
                         jnp.concatenate(
                             [jnp.ones((Bn, 1), dtype=bool), joker_real],
                             axis=1),
                         jnp.concatenate(
                             [joker_real, jnp.zeros((Bn, 1), dtype=bool)],
                             axis=1))
    no_mod = ~jnp.any(mod_mask, axis=1)
    mod_mask = mod_mask.at[:, 0].set(mod_mask[:, 0] | no_mod)

    return (hand_toks, hmask, run_tok, ctx_seq, ctx_mask, mod_seq, mod_mask)
